# Initial kernel scaffold; baseline (speedup 1.0000x reference)
import jax
import jax.numpy as jnp
from jax.experimental import pallas as pl


def kernel(input_ids, attention_mask, emb_weight, proj_weight, proj_bias):
    raise NotImplementedError("write your pallas kernel here")


# speedup vs baseline: 116.5531x; 116.5531x over previous
import jax
import jax.numpy as jnp
from jax import lax
from jax.experimental import pallas as pl
from jax.experimental.pallas import tpu as pltpu
from jax.experimental.pallas import tpu_sc as plsc

B = 16384
L = 200
V = 512
D = 16
NW = 32
RPW = B // NW
CHUNK = 32
NCH = RPW // CHUNK
GP = CHUNK // 16

_mesh = plsc.VectorSubcoreMesh(core_axis_name="c", subcore_axis_name="s")

_OUT_TYPE = jax.ShapeDtypeStruct((B,), jnp.float32)
_SCRATCH = [
    pltpu.VMEM((CHUNK * L,), jnp.int32),
    pltpu.VMEM((CHUNK * L,), jnp.int32),
    pltpu.VMEM((CHUNK * L,), jnp.int32),
    pltpu.VMEM((CHUNK * L,), jnp.int32),
    pltpu.VMEM((D * V,), jnp.float32),
    pltpu.VMEM((D * 16,), jnp.float32),
    pltpu.VMEM((16,), jnp.float32),
    pltpu.VMEM((16 * V,), jnp.float32),
    pltpu.VMEM((RPW,), jnp.float32),
    pltpu.SemaphoreType.DMA,
    pltpu.SemaphoreType.DMA,
]


def _sc_body(ids_hbm, mask_hbm, embt_hbm, wb_hbm, bias_hbm, out_hbm,
             ids_a, ids_b, mask_a, mask_b, embt_v, wb_v, bias_v, s_v, out_v,
             sem_a, sem_b):
    wid = lax.axis_index("s") * 2 + lax.axis_index("c")
    base = wid * RPW
    lanes = lax.iota(jnp.int32, 16)

    pltpu.sync_copy(embt_hbm, embt_v)
    pltpu.sync_copy(wb_hbm, wb_v)
    pltpu.sync_copy(bias_hbm, bias_v)

    wvecs = [wb_v[pl.ds(d * 16, 16)] for d in range(D)]
    for g in range(V // 16):
        acc = jnp.zeros((16,), jnp.float32)
        for d in range(D):
            acc = acc + embt_v[pl.ds(d * V + g * 16, 16)] * wvecs[d]
        for j in range(16):
            s_v[pl.ds(j * V + g * 16, 16)] = acc

    bias_vec = bias_v[...]
    lane_off = lanes * V

    def start(ch, idbuf, mkbuf, sem):
        e0 = (base + ch * CHUNK) * L
        h1 = pltpu.async_copy(ids_hbm.at[pl.ds(e0, CHUNK * L)], idbuf, sem)
        h2 = pltpu.async_copy(mask_hbm.at[pl.ds(e0, CHUNK * L)], mkbuf, sem)
        return h1, h2

    bufs = [(ids_a, mask_a, sem_a), (ids_b, mask_b, sem_b)]
    pending = start(0, *bufs[0])
    for ch in range(NCH):
        nxt = start(ch + 1, *bufs[(ch + 1) % 2]) if ch + 1 < NCH else None
        pending[0].wait()
        pending[1].wait()
        idbuf, mkbuf, _ = bufs[ch % 2]
        for g in range(GP):
            pos0 = (lanes + g * 16) * L

            def body(l, carry):
                acc, cnt, pos = carry
                idv = plsc.load_gather(idbuf, [pos])
                mv = plsc.load_gather(mkbuf, [pos])
                sval = plsc.load_gather(s_v, [idv + lane_off])
                acc = acc + sval * mv.astype(jnp.float32)
                return acc, cnt + mv, pos + 1

            acc, cnt, _ = lax.fori_loop(
                0, L, body,
                (jnp.zeros((16,), jnp.float32), jnp.zeros((16,), jnp.int32),
                 pos0))
            denom = jnp.maximum(cnt.astype(jnp.float32), 1.0)
            out_v[pl.ds((ch * GP + g) * 16, 16)] = acc / denom + bias_vec
        pending = nxt

    pltpu.sync_copy(out_v, out_hbm.at[pl.ds(base, RPW)])


_sc_pool = pl.kernel(
    _sc_body,
    out_type=_OUT_TYPE,
    mesh=_mesh,
    compiler_params=pltpu.CompilerParams(needs_layout_passes=False),
    scratch_types=_SCRATCH,
)


def kernel(input_ids, attention_mask, emb_weight, proj_weight, proj_bias):
    ids = input_ids.astype(jnp.int32).reshape(B * L)
    mask = attention_mask.astype(jnp.int32).reshape(B * L)
    embt = emb_weight.astype(jnp.float32).T.reshape(D * V)
    wb = jnp.broadcast_to(
        proj_weight.astype(jnp.float32).reshape(D, 1), (D, 16)).reshape(D * 16)
    bias = jnp.broadcast_to(proj_bias.astype(jnp.float32), (16,))
    out = _sc_pool(ids, mask, embt, wb, bias)
    return out.reshape(B, 1)


# speedup vs baseline: 125.6520x; 1.0781x over previous
import jax
import jax.numpy as jnp
from jax import lax
from jax.experimental import pallas as pl
from jax.experimental.pallas import tpu as pltpu
from jax.experimental.pallas import tpu_sc as plsc

B = 16384
L = 200
V = 512
D = 16
NW = 32
RPW = B // NW
CHUNK = 32
NCH = RPW // CHUNK
GP = CHUNK // 16
UNROLL = 8

_mesh = plsc.VectorSubcoreMesh(core_axis_name="c", subcore_axis_name="s")

_OUT_TYPE = jax.ShapeDtypeStruct((B,), jnp.float32)
_SCRATCH = [
    pltpu.VMEM((CHUNK * L,), jnp.int32),
    pltpu.VMEM((CHUNK * L,), jnp.int32),
    pltpu.VMEM((CHUNK * L,), jnp.int32),
    pltpu.VMEM((CHUNK * L,), jnp.int32),
    pltpu.VMEM((D * V,), jnp.float32),
    pltpu.VMEM((D * 16,), jnp.float32),
    pltpu.VMEM((16,), jnp.float32),
    pltpu.VMEM((16 * V,), jnp.float32),
    pltpu.VMEM((RPW,), jnp.float32),
    pltpu.SemaphoreType.DMA,
    pltpu.SemaphoreType.DMA,
]


def _sc_body(ids_hbm, mask_hbm, embt_hbm, wb_hbm, bias_hbm, out_hbm,
             ids_a, ids_b, mask_a, mask_b, embt_v, wb_v, bias_v, s_v, out_v,
             sem_a, sem_b):
    wid = lax.axis_index("s") * 2 + lax.axis_index("c")
    base = wid * RPW
    lanes = lax.iota(jnp.int32, 16)

    pltpu.sync_copy(embt_hbm, embt_v)
    pltpu.sync_copy(wb_hbm, wb_v)
    pltpu.sync_copy(bias_hbm, bias_v)

    wvecs = [wb_v[pl.ds(d * 16, 16)] for d in range(D)]
    for g in range(V // 16):
        acc = jnp.zeros((16,), jnp.float32)
        for d in range(D):
            acc = acc + embt_v[pl.ds(d * V + g * 16, 16)] * wvecs[d]
        for j in range(16):
            s_v[pl.ds(j * V + g * 16, 16)] = acc

    bias_vec = bias_v[...]
    lane_off = lanes * V

    def start(ch, idbuf, mkbuf, sem):
        e0 = (base + ch * CHUNK) * L
        h1 = pltpu.async_copy(ids_hbm.at[pl.ds(e0, CHUNK * L)], idbuf, sem)
        h2 = pltpu.async_copy(mask_hbm.at[pl.ds(e0, CHUNK * L)], mkbuf, sem)
        return h1, h2

    bufs = [(ids_a, mask_a, sem_a), (ids_b, mask_b, sem_b)]
    pending = start(0, *bufs[0])
    for ch in range(NCH):
        nxt = start(ch + 1, *bufs[(ch + 1) % 2]) if ch + 1 < NCH else None
        pending[0].wait()
        pending[1].wait()
        idbuf, mkbuf, _ = bufs[ch % 2]
        for g in range(GP):
            pos0 = (lanes + g * 16) * L

            def body(it, carry):
                acc, cnt, pos = carry
                for k in range(UNROLL):
                    p = pos + k
                    idv = plsc.load_gather(idbuf, [p])
                    mv = plsc.load_gather(mkbuf, [p])
                    sval = plsc.load_gather(s_v, [idv + lane_off])
                    acc = acc + sval * mv.astype(jnp.float32)
                    cnt = cnt + mv
                return acc, cnt, pos + UNROLL

            acc, cnt, _ = lax.fori_loop(
                0, L // UNROLL, body,
                (jnp.zeros((16,), jnp.float32), jnp.zeros((16,), jnp.int32),
                 pos0))
            denom = jnp.maximum(cnt.astype(jnp.float32), 1.0)
            inv = 1.0 / denom
            inv = inv * (2.0 - denom * inv)
            out_v[pl.ds((ch * GP + g) * 16, 16)] = acc * inv + bias_vec
        pending = nxt

    pltpu.sync_copy(out_v, out_hbm.at[pl.ds(base, RPW)])


_sc_pool = pl.kernel(
    _sc_body,
    out_type=_OUT_TYPE,
    mesh=_mesh,
    compiler_params=pltpu.CompilerParams(needs_layout_passes=False),
    scratch_types=_SCRATCH,
)


def kernel(input_ids, attention_mask, emb_weight, proj_weight, proj_bias):
    ids = input_ids.astype(jnp.int32).reshape(B * L)
    mask = attention_mask.astype(jnp.int32).reshape(B * L)
    embt = emb_weight.astype(jnp.float32).T.reshape(D * V)
    wb = jnp.broadcast_to(
        proj_weight.astype(jnp.float32).reshape(D, 1), (D, 16)).reshape(D * 16)
    bias = jnp.broadcast_to(proj_bias.astype(jnp.float32), (16,))
    out = _sc_pool(ids, mask, embt, wb, bias)
    return out.reshape(B, 1)
